# baseline (speedup 1.0000x reference)
import functools

import jax
import jax.numpy as jnp
from jax import lax
from jax.experimental import pallas as pl
from jax.experimental.pallas import tpu as pltpu
from jax.experimental.pallas import tpu_sc as plsc
from jax._src.pallas import mpmd as _mpmd

N = 100000
D = 500
DP = 512
B = 4096
TD = 100
HID = 550
MSGD = 100

NC = 2
NS = 16
NW = NC * NS
BPW = B // NW

_BB = 256
_NB = B // _BB


def _sc_mesh():
  return plsc.VectorSubcoreMesh(core_axis_name="c", subcore_axis_name="s",
                                num_cores=NC, num_subcores=NS)


def _gather_body(memp_hbm, lu_hbm, src_hbm, dst_hbm,
                 srcm_out, dstm_out, lus_out,
                 idx_v, rows_v, lu_v, sem):
  wid = lax.axis_index("s") * NC + lax.axis_index("c")
  base = wid * BPW
  pltpu.sync_copy(src_hbm.at[pl.ds(base, BPW)], idx_v)
  pltpu.async_copy(memp_hbm.at[idx_v], rows_v, sem).wait()
  pltpu.sync_copy(rows_v, srcm_out.at[pl.ds(base, BPW)])
  pltpu.async_copy(lu_hbm.at[idx_v], lu_v, sem).wait()
  pltpu.sync_copy(lu_v, lus_out.at[pl.ds(base, BPW)])
  pltpu.sync_copy(dst_hbm.at[pl.ds(base, BPW)], idx_v)
  pltpu.async_copy(memp_hbm.at[idx_v], rows_v, sem).wait()
  pltpu.sync_copy(rows_v, dstm_out.at[pl.ds(base, BPW)])


@functools.cache
def _make_sc_gather():
  return pl.kernel(
      _gather_body,
      out_type=(
          jax.ShapeDtypeStruct((B, DP), jnp.float32),
          jax.ShapeDtypeStruct((B, DP), jnp.float32),
          jax.ShapeDtypeStruct((B,), jnp.float32),
      ),
      mesh=_sc_mesh(),
      scratch_types=[
          pltpu.VMEM((BPW,), jnp.int32),
          pltpu.VMEM((BPW, DP), jnp.float32),
          pltpu.VMEM((BPW,), jnp.float32),
          pltpu.SemaphoreType.DMA,
      ],
  )


def _sigmoid(x):
  return 1.0 / (1.0 + jnp.exp(-x))


def _dense_body(srcm_ref, dstm_ref, et_ref, lus_ref, tw_ref, tb_ref,
                w1s_ref, w1d_ref, w1t_ref, b1_ref, w2_ref, b2_ref,
                wir_ref, wiz_ref, win_ref, whr_ref, whz_ref, whn_ref,
                bi_ref, bh_ref, srcall_ref, h_ref, tl_ref):
  src_mem = srcm_ref[...]
  dst_mem = dstm_ref[...]
  dt = et_ref[0, 0, :] - lus_ref[0, 0, :]
  tenc = jnp.cos(dt[:, None] * tw_ref[0, :][None, :] + tb_ref[0, :][None, :])
  h = (jnp.dot(src_mem, w1s_ref[...], preferred_element_type=jnp.float32)
       + jnp.dot(dst_mem, w1d_ref[...], preferred_element_type=jnp.float32)
       + jnp.dot(tenc, w1t_ref[...], preferred_element_type=jnp.float32)
       + b1_ref[0, :][None, :])
  h = jnp.maximum(h, 0.0)
  msg = jnp.dot(h, w2_ref[...], preferred_element_type=jnp.float32) \
      + b2_ref[0, :][None, :]
  i_r = jnp.dot(msg, wir_ref[...], preferred_element_type=jnp.float32) \
      + bi_ref[0, :][None, :]
  i_z = jnp.dot(msg, wiz_ref[...], preferred_element_type=jnp.float32) \
      + bi_ref[1, :][None, :]
  i_n = jnp.dot(msg, win_ref[...], preferred_element_type=jnp.float32) \
      + bi_ref[2, :][None, :]
  h_r = jnp.dot(src_mem, whr_ref[...], preferred_element_type=jnp.float32) \
      + bh_ref[0, :][None, :]
  h_z = jnp.dot(src_mem, whz_ref[...], preferred_element_type=jnp.float32) \
      + bh_ref[1, :][None, :]
  h_n = jnp.dot(src_mem, whn_ref[...], preferred_element_type=jnp.float32) \
      + bh_ref[2, :][None, :]
  r = _sigmoid(i_r + h_r)
  z = _sigmoid(i_z + h_z)
  n = jnp.tanh(i_n + r * h_n)
  h_new = (1.0 - z) * n + z * src_mem[:, :D]
  h_ref[...] = jnp.concatenate(
      [h_new, jnp.zeros((_BB, DP - D), jnp.float32)], axis=1)

  blk = pl.program_id(0)
  src_all = srcall_ref[0, :]
  src_blk = srcall_ref[0, pl.ds(blk * _BB, _BB)]
  eq = src_blk[:, None] == src_all[None, :]
  jj = lax.broadcasted_iota(jnp.int32, (_BB, B), 1)
  tl_ref[0, 0, :] = jnp.max(jnp.where(eq, jj, -1), axis=1)


def _tc_dense(src_mem, dst_mem, edge_times, lu_src, time_w, time_b,
              w1s, w1d, w1t, b1, w2, b2, wir, wiz, win, whr, whz, whn,
              b_ih, b_hh, src_idx):
  et3 = edge_times.reshape(_NB, 1, _BB)
  lus3 = lu_src.reshape(_NB, 1, _BB)
  src2 = src_idx.reshape(1, B)
  full = lambda shape: pl.BlockSpec(shape, lambda i: (0,) * len(shape))
  h_new, tl3 = pl.pallas_call(
      _dense_body,
      grid=(_NB,),
      in_specs=[
          pl.BlockSpec((_BB, DP), lambda i: (i, 0)),
          pl.BlockSpec((_BB, DP), lambda i: (i, 0)),
          pl.BlockSpec((1, 1, _BB), lambda i: (i, 0, 0)),
          pl.BlockSpec((1, 1, _BB), lambda i: (i, 0, 0)),
          full((1, TD)), full((1, TD)),
          full((DP, HID)), full((DP, HID)), full((TD, HID)), full((1, HID)),
          full((HID, MSGD)), full((1, MSGD)),
          full((MSGD, D)), full((MSGD, D)), full((MSGD, D)),
          full((DP, D)), full((DP, D)), full((DP, D)),
          full((3, D)), full((3, D)),
          full((1, B)),
      ],
      out_specs=[
          pl.BlockSpec((_BB, DP), lambda i: (i, 0)),
          pl.BlockSpec((1, 1, _BB), lambda i: (i, 0, 0)),
      ],
      out_shape=[
          jax.ShapeDtypeStruct((B, DP), jnp.float32),
          jax.ShapeDtypeStruct((_NB, 1, _BB), jnp.int32),
      ],
  )(src_mem, dst_mem, et3, lus3, time_w.reshape(1, TD), time_b.reshape(1, TD),
    w1s, w1d, w1t, b1.reshape(1, HID), w2, b2.reshape(1, MSGD),
    wir, wiz, win, whr, whz, whn,
    b_ih.reshape(3, D), b_hh.reshape(3, D), src2)
  return h_new, tl3.reshape(B)


def _scatter_body(memp_in, lu_in, hnew_hbm, et_hbm, src_hbm, tl_hbm,
                  memp_out, lu_out,
                  srcv, tlv, rows_v, etv, sem):
  del memp_in, lu_in
  wid = lax.axis_index("s") * NC + lax.axis_index("c")
  base = wid * BPW
  pltpu.sync_copy(src_hbm.at[pl.ds(base, BPW)], srcv)
  pltpu.sync_copy(tl_hbm.at[pl.ds(base, BPW)], tlv)
  pltpu.async_copy(hnew_hbm.at[tlv], rows_v, sem).wait()
  pltpu.async_copy(et_hbm.at[tlv], etv, sem).wait()
  pltpu.async_copy(rows_v, memp_out.at[srcv], sem).wait()
  pltpu.async_copy(etv, lu_out.at[srcv], sem).wait()


@functools.cache
def _make_sc_scatter():
  return _mpmd._mpmd_map(
      [(_sc_mesh(), _scatter_body)],
      (
          jax.ShapeDtypeStruct((N, DP), jnp.float32),
          jax.ShapeDtypeStruct((N,), jnp.float32),
      ),
      input_output_aliases={0: 0, 1: 1},
      scratch_types=[
          pltpu.VMEM((BPW,), jnp.int32),
          pltpu.VMEM((BPW,), jnp.int32),
          pltpu.VMEM((BPW, DP), jnp.float32),
          pltpu.VMEM((BPW,), jnp.float32),
          pltpu.SemaphoreType.DMA,
      ],
      compiler_params=None,
      interpret=False,
      debug=False,
      cost_estimate=None,
      name="sc_scatter",
      metadata=None,
  )


def kernel(memory, last_update, edge_times, time_w, time_b,
           W1, b1, W2, b2, W_ih, W_hh, b_ih, b_hh, src_idx, dst_idx):
  pad = lambda w: jnp.concatenate(
      [w, jnp.zeros((DP - D,) + w.shape[1:], w.dtype)], axis=0)
  w1s, w1d, w1t = pad(W1[:D]), pad(W1[D:2 * D]), W1[2 * D:]
  wih_t = W_ih.T
  whh_t = W_hh.T
  wir, wiz, win = wih_t[:, :D], wih_t[:, D:2 * D], wih_t[:, 2 * D:]
  whr, whz, whn = (pad(whh_t[:, :D]), pad(whh_t[:, D:2 * D]),
                   pad(whh_t[:, 2 * D:]))

  memp = jnp.pad(memory, ((0, 0), (0, DP - D)))
  lu0 = last_update * 1.0

  src_mem, dst_mem, lu_src = _make_sc_gather()(
      memp, last_update, src_idx, dst_idx)

  h_new, tl = _tc_dense(
      src_mem, dst_mem, edge_times, lu_src, time_w, time_b,
      w1s, w1d, w1t, b1, W2, b2, wir, wiz, win, whr, whz, whn,
      b_ih, b_hh, src_idx)

  new_memp, new_lu = _make_sc_scatter()(
      memp, lu0, h_new, edge_times, src_idx, tl)
  return new_memp[:, :D], new_lu
